# baseline (speedup 1.0000x reference)
import functools

import jax
import jax.numpy as jnp
from jax import lax
from jax.experimental import pallas as pl
from jax.experimental.pallas import tpu as pltpu
from jax.experimental.pallas import tpu_sc as plsc

N0 = 2048
D = 256
K0 = 1843
K1 = 1290
NP1 = 1920
NP2 = 1408
B1 = 1536


def _maskify_body(g_ref, a_ref, at_ref):
    a = (g_ref[...] != 0).astype(jnp.bfloat16)
    a_ref[...] = a
    at_ref[...] = a.T


def _maskify(g, bm=256):
    n = g.shape[0]
    return pl.pallas_call(
        _maskify_body,
        grid=(n // bm, n // bm),
        in_specs=[pl.BlockSpec((bm, bm), lambda i, j: (i, j))],
        out_specs=[pl.BlockSpec((bm, bm), lambda i, j: (i, j)),
                   pl.BlockSpec((bm, bm), lambda i, j: (j, i))],
        out_shape=[jax.ShapeDtypeStruct((n, n), jnp.bfloat16),
                   jax.ShapeDtypeStruct((n, n), jnp.bfloat16)],
    )(g)


def _transpose_body(x_ref, o_ref):
    o_ref[...] = x_ref[...].T


def _transpose(x, bm=128):
    n = x.shape[0]
    return pl.pallas_call(
        _transpose_body,
        grid=(n // bm, n // bm),
        in_specs=[pl.BlockSpec((bm, bm), lambda i, j: (i, j))],
        out_specs=pl.BlockSpec((bm, bm), lambda i, j: (j, i)),
        out_shape=jax.ShapeDtypeStruct((n, n), x.dtype),
    )(x)


def _make_sc_gather(V, W, B, dtype):
    info = plsc.get_sparse_core_info()
    nw = info.num_cores * info.num_subcores
    num_cores = info.num_cores
    b_per_w = B // nw
    rows_per_chunk = b_per_w
    while rows_per_chunk * W > 120_000:
        rows_per_chunk //= 2
    nchunks = b_per_w // rows_per_chunk
    mesh = plsc.VectorSubcoreMesh(core_axis_name="c", subcore_axis_name="s")

    @functools.partial(
        pl.kernel, mesh=mesh,
        out_type=jax.ShapeDtypeStruct((B, W), dtype),
        scratch_types=[
            pltpu.VMEM((rows_per_chunk,), jnp.int32),
            pltpu.VMEM((rows_per_chunk, W), dtype),
            pltpu.SemaphoreType.DMA,
        ],
    )
    def gather_k(table_hbm, idx_hbm, out_hbm, idx_v, rows_v, sem):
        wid = lax.axis_index("s") * num_cores + lax.axis_index("c")
        base = wid * b_per_w
        for c in range(nchunks):
            off = base + c * rows_per_chunk
            pltpu.sync_copy(idx_hbm.at[pl.ds(off, rows_per_chunk)], idx_v)
            pltpu.async_copy(table_hbm.at[idx_v], rows_v, sem).wait()
            pltpu.sync_copy(rows_v, out_hbm.at[pl.ds(off, rows_per_chunk)])

    return gather_k


@functools.lru_cache(maxsize=None)
def _sc_gather_fn(V, W, B, dtype):
    return _make_sc_gather(V, W, B, dtype)


def _sc_gather(table, idx):
    V, W = table.shape
    B = idx.shape[0]
    return _sc_gather_fn(V, W, B, table.dtype)(table, idx)


def _i32view(x):
    n, m = x.shape
    return lax.bitcast_convert_type(x.reshape(n, m // 2, 2), jnp.int32)


def _bf16view(y):
    n, m = y.shape
    return lax.bitcast_convert_type(y, jnp.bfloat16).reshape(n, 2 * m)


def _gather_bf16(table, idx):
    return _bf16view(_sc_gather(_i32view(table), idx))


def _mask_body(kv_ref, b_ref, c_ref, *outs, bm, emit_m):
    kv = kv_ref[0]
    i = pl.program_id(0)
    s = lax.dot_general(b_ref[...], c_ref[...], (((1,), (1,)), ((), ())),
                        preferred_element_type=jnp.float32)
    rows = jax.lax.broadcasted_iota(jnp.int32, s.shape, 0) + i * bm
    cols = jax.lax.broadcasted_iota(jnp.int32, s.shape, 1)
    valid = (rows < kv) & (cols < kv)
    m = jnp.where((s > 0) & valid, 1.0, 0.0)
    deg = jnp.sum(m, axis=1, keepdims=True)
    gn = m / jnp.maximum(deg, 1.0)
    if emit_m:
        m_ref, gn_ref = outs
        m_ref[...] = m.astype(jnp.bfloat16)
    else:
        (gn_ref,) = outs
    gn_ref[...] = gn


def _mask_level(bg, ct, kv, M, N, emit_m, bm=128):
    K = bg.shape[1]
    body = functools.partial(_mask_body, bm=bm, emit_m=emit_m)
    out_specs = [pl.BlockSpec((bm, N), lambda i: (i, 0))]
    out_shape = [jax.ShapeDtypeStruct((M, N), jnp.float32)]
    if emit_m:
        out_specs = [pl.BlockSpec((bm, N), lambda i: (i, 0))] + out_specs
        out_shape = [jax.ShapeDtypeStruct((M, N), jnp.bfloat16)] + out_shape
    return pl.pallas_call(
        body,
        grid=(M // bm,),
        in_specs=[pl.BlockSpec(memory_space=pltpu.SMEM),
                  pl.BlockSpec((bm, K), lambda i: (i, 0)),
                  pl.BlockSpec((N, K), lambda i: (0, 0))],
        out_specs=out_specs,
        out_shape=out_shape,
    )(jnp.full((1,), kv, jnp.int32), bg, ct)


def _gcn_body(kv_ref, g_ref, x_ref, sc_ref, w_ref, b_ref, *rest, bm, with_res):
    if with_res:
        res_ref, o_ref = rest
    else:
        (o_ref,) = rest
    kv = kv_ref[0]
    i = pl.program_id(0)
    x = x_ref[...] * sc_ref[...]
    p = jnp.dot(g_ref[...], x, preferred_element_type=jnp.float32)
    y = jax.nn.relu(jnp.dot(p, w_ref[...], preferred_element_type=jnp.float32)
                    + b_ref[...])
    if with_res:
        y = y + res_ref[...]
    rows = jax.lax.broadcasted_iota(jnp.int32, y.shape, 0) + i * bm
    o_ref[...] = jnp.where(rows < kv, y, 0.0)


def _gcn(g, x, scale, w, b, kv, res=None, bm=128):
    n = g.shape[0]
    body = functools.partial(_gcn_body, bm=bm, with_res=res is not None)
    in_specs = [pl.BlockSpec(memory_space=pltpu.SMEM),
                pl.BlockSpec((bm, n), lambda i: (i, 0)),
                pl.BlockSpec((n, D), lambda i: (0, 0)),
                pl.BlockSpec((n, 1), lambda i: (0, 0)),
                pl.BlockSpec((D, D), lambda i: (0, 0)),
                pl.BlockSpec((1, D), lambda i: (0, 0))]
    args = [jnp.full((1,), kv, jnp.int32), g, x, scale, w, b.reshape(1, D)]
    if res is not None:
        in_specs.append(pl.BlockSpec((bm, D), lambda i: (i, 0)))
        args.append(res)
    return pl.pallas_call(
        body,
        grid=(n // bm,),
        in_specs=in_specs,
        out_specs=pl.BlockSpec((bm, D), lambda i: (i, 0)),
        out_shape=jax.ShapeDtypeStruct((n, D), jnp.float32),
    )(*args)


def _gcn_final_body(g_ref, x_ref, w_ref, b_ref, res_ref, h0_ref, o1_ref, o2_ref):
    p = jnp.dot(g_ref[...], x_ref[...], preferred_element_type=jnp.float32)
    y = jax.nn.relu(jnp.dot(p, w_ref[...], preferred_element_type=jnp.float32)
                    + b_ref[...]) + res_ref[...]
    o1_ref[...] = y
    o2_ref[...] = y + h0_ref[...]


def _gcn_final(g, x, w, b, res, h0, bm=256):
    n = g.shape[0]
    return pl.pallas_call(
        _gcn_final_body,
        grid=(n // bm,),
        in_specs=[pl.BlockSpec((bm, n), lambda i: (i, 0)),
                  pl.BlockSpec((n, D), lambda i: (0, 0)),
                  pl.BlockSpec((D, D), lambda i: (0, 0)),
                  pl.BlockSpec((1, D), lambda i: (0, 0)),
                  pl.BlockSpec((bm, D), lambda i: (i, 0)),
                  pl.BlockSpec((bm, D), lambda i: (i, 0))],
        out_specs=[pl.BlockSpec((bm, D), lambda i: (i, 0)),
                   pl.BlockSpec((bm, D), lambda i: (i, 0))],
        out_shape=[jax.ShapeDtypeStruct((n, D), jnp.float32),
                   jax.ShapeDtypeStruct((n, D), jnp.float32)],
    )(g, x, w, b.reshape(1, D), res, h0)


def _pad_idx(idx, b):
    return jnp.concatenate([idx.astype(jnp.int32),
                            jnp.zeros((b - idx.shape[0],), jnp.int32)])


def kernel(g, h, Wd0, bd0, Wd1, bd1, Wb, bb, Wu0, bu0, Wu1, bu1,
           Wp0, bp0, Wp1, bp1):
    h1 = jax.nn.relu(jnp.matmul(g, h) @ Wd0 + bd0)
    s0 = jax.nn.sigmoid((h1 @ Wp0 + bp0)[:, 0])
    v0, idx0 = lax.top_k(s0, K0)
    idx0p = _pad_idx(idx0, N0)

    a0, a0t = _maskify(g)
    b0g = _gather_bf16(a0, idx0p)
    c0t = _gather_bf16(a0t, idx0p)
    a1, g1 = _mask_level(b0g, c0t, K0, N0, N0, emit_m=True)

    newh_raw = _sc_gather(h1, idx0p)
    newh = newh_raw[:K0] * v0[:, None]
    g1c = g1[:K0, :K0]
    h2 = jax.nn.relu(jnp.matmul(g1c, newh) @ Wd1 + bd1)
    s1 = jax.nn.sigmoid((h2 @ Wp1 + bp1)[:, 0])
    v1k, idx1 = lax.top_k(s1, K1)
    idx1p = _pad_idx(idx1, B1)

    a1t = _transpose(a1, bm=256)
    b1g = _gather_bf16(a1, idx1p)
    c1t = _gather_bf16(a1t, idx1p)
    g2 = _mask_level(b1g, c1t, K1, NP2, NP2, emit_m=False)[0]

    h2p = jnp.zeros((N0, D), jnp.float32).at[:K0].set(h2)
    newh2_raw = _sc_gather(h2p, idx1p)
    v1p = jnp.zeros((NP2, 1), jnp.float32).at[:K1, 0].set(v1k)
    hb = _gcn(g2, newh2_raw[:NP2], v1p, Wb, bb, K1)

    u1idx = jnp.full((N0,), K1, jnp.int32).at[idx1].set(
        jnp.arange(K1, dtype=jnp.int32))
    u1 = _sc_gather(hb, u1idx)

    ones = jnp.ones((N0, 1), jnp.float32)
    x1 = _gcn(g1, u1, ones, Wu0, bu0, K0, res=h2p, bm=256)

    u0idx = jnp.full((N0,), K0, jnp.int32).at[idx0].set(
        jnp.arange(K0, dtype=jnp.int32))
    u0 = _sc_gather(x1, u0idx)

    y1, y2 = _gcn_final(g, u0, Wu1, bu1, h1, h)

    return (x1[:K0], y1, y2)


# speedup vs baseline: 2.1056x; 2.1056x over previous
import functools

import jax
import jax.numpy as jnp
from jax import lax
from jax.experimental import pallas as pl
from jax.experimental.pallas import tpu as pltpu
from jax.experimental.pallas import tpu_sc as plsc

N0 = 2048
D = 256
K0 = 1843
K1 = 1290


def _maskify_body(g_ref, a_ref, at_ref):
    a = (g_ref[...] != 0).astype(jnp.float32)
    a_ref[...] = a
    at_ref[...] = a.T


def _maskify(g, bm=256):
    n = g.shape[0]
    return pl.pallas_call(
        _maskify_body,
        grid=(n // bm, n // bm),
        in_specs=[pl.BlockSpec((bm, bm), lambda i, j: (i, j))],
        out_specs=[pl.BlockSpec((bm, bm), lambda i, j: (i, j)),
                   pl.BlockSpec((bm, bm), lambda i, j: (j, i))],
        out_shape=[jax.ShapeDtypeStruct((n, n), jnp.float32),
                   jax.ShapeDtypeStruct((n, n), jnp.float32)],
    )(g)


def _transpose_body(x_ref, o_ref):
    o_ref[...] = x_ref[...].T


def _transpose(x, bm=256):
    n = x.shape[0]
    return pl.pallas_call(
        _transpose_body,
        grid=(n // bm, n // bm),
        in_specs=[pl.BlockSpec((bm, bm), lambda i, j: (i, j))],
        out_specs=pl.BlockSpec((bm, bm), lambda i, j: (j, i)),
        out_shape=jax.ShapeDtypeStruct((n, n), x.dtype),
    )(x)


def _make_sc_gather(V, W, B):
    info = plsc.get_sparse_core_info()
    nw = info.num_cores * info.num_subcores
    num_cores = info.num_cores
    b_per_w = B // nw
    rows_per_chunk = b_per_w
    while rows_per_chunk * W > 120_000:
        rows_per_chunk //= 2
    nchunks = b_per_w // rows_per_chunk
    mesh = plsc.VectorSubcoreMesh(core_axis_name="c", subcore_axis_name="s")

    @functools.partial(
        pl.kernel, mesh=mesh,
        out_type=jax.ShapeDtypeStruct((B, W), jnp.float32),
        scratch_types=[
            pltpu.VMEM((rows_per_chunk,), jnp.int32),
            pltpu.VMEM((rows_per_chunk, W), jnp.float32),
            pltpu.SemaphoreType.DMA,
        ],
    )
    def gather_k(table_hbm, idx_hbm, out_hbm, idx_v, rows_v, sem):
        wid = lax.axis_index("s") * num_cores + lax.axis_index("c")
        base = wid * b_per_w
        for c in range(nchunks):
            off = base + c * rows_per_chunk
            pltpu.sync_copy(idx_hbm.at[pl.ds(off, rows_per_chunk)], idx_v)
            pltpu.async_copy(table_hbm.at[idx_v], rows_v, sem).wait()
            pltpu.sync_copy(rows_v, out_hbm.at[pl.ds(off, rows_per_chunk)])

    return gather_k


@functools.lru_cache(maxsize=None)
def _sc_gather_fn(V, W, B):
    return _make_sc_gather(V, W, B)


def _sc_gather(table, idx):
    V, W = table.shape
    B = idx.shape[0]
    return _sc_gather_fn(V, W, B)(table, idx)


def _make_sc_gather3(V, W, Wh, B, chunk):
    info = plsc.get_sparse_core_info()
    nw = info.num_cores * info.num_subcores
    num_cores = info.num_cores
    b_per_w = B // nw
    nchunks = b_per_w // chunk
    mesh = plsc.VectorSubcoreMesh(core_axis_name="c", subcore_axis_name="s")

    @functools.partial(
        pl.kernel, mesh=mesh,
        out_type=[jax.ShapeDtypeStruct((B, W), jnp.float32),
                  jax.ShapeDtypeStruct((B, W), jnp.float32),
                  jax.ShapeDtypeStruct((B, Wh), jnp.float32)],
        scratch_types=[
            pltpu.VMEM((b_per_w,), jnp.int32),
            pltpu.VMEM((chunk, W), jnp.float32),
            pltpu.VMEM((chunk, W), jnp.float32),
            pltpu.VMEM((b_per_w, Wh), jnp.float32),
            pltpu.SemaphoreType.DMA,
            pltpu.SemaphoreType.DMA,
            pltpu.SemaphoreType.DMA,
        ],
    )
    def gather_k(t0_hbm, t1_hbm, th_hbm, idx_hbm, o0_hbm, o1_hbm, oh_hbm,
                 idx_v, r0, r1, rh, s0, s1, sh):
        wid = lax.axis_index("s") * num_cores + lax.axis_index("c")
        base = wid * b_per_w
        pltpu.sync_copy(idx_hbm.at[pl.ds(base, b_per_w)], idx_v)
        ch = pltpu.async_copy(th_hbm.at[idx_v], rh, sh)
        for c in range(nchunks):
            off = base + c * chunk
            sl = pl.ds(c * chunk, chunk)
            c0 = pltpu.async_copy(t0_hbm.at[idx_v.at[sl]], r0, s0)
            c1 = pltpu.async_copy(t1_hbm.at[idx_v.at[sl]], r1, s1)
            c0.wait()
            pltpu.sync_copy(r0, o0_hbm.at[pl.ds(off, chunk)])
            c1.wait()
            pltpu.sync_copy(r1, o1_hbm.at[pl.ds(off, chunk)])
        ch.wait()
        pltpu.sync_copy(rh, oh_hbm.at[pl.ds(base, b_per_w)])

    return gather_k


@functools.lru_cache(maxsize=None)
def _sc_gather3_fn(V, W, Wh, B, chunk):
    return _make_sc_gather3(V, W, Wh, B, chunk)


def _sc_gather3(t0, t1, th, idx, chunk=16):
    V, W = t0.shape
    Wh = th.shape[1]
    B = idx.shape[0]
    return _sc_gather3_fn(V, W, Wh, B, chunk)(t0, t1, th, idx)


def _mask_body(kv_ref, b_ref, c_ref, m_ref, gn_ref, *, bm):
    kv = kv_ref[0]
    i = pl.program_id(0)
    s = lax.dot_general(b_ref[...], c_ref[...], (((1,), (1,)), ((), ())),
                        preferred_element_type=jnp.float32)
    rows = jax.lax.broadcasted_iota(jnp.int32, s.shape, 0) + i * bm
    cols = jax.lax.broadcasted_iota(jnp.int32, s.shape, 1)
    valid = (rows < kv) & (cols < kv)
    m = jnp.where((s > 0) & valid, 1.0, 0.0)
    m_ref[...] = m
    deg = jnp.sum(m, axis=1, keepdims=True)
    gn_ref[...] = m / jnp.maximum(deg, 1.0)


def _mask_level(bg, ct, kv, bm=256):
    n = bg.shape[0]
    body = functools.partial(_mask_body, bm=bm)
    return pl.pallas_call(
        body,
        grid=(n // bm,),
        in_specs=[pl.BlockSpec(memory_space=pltpu.SMEM),
                  pl.BlockSpec((bm, n), lambda i: (i, 0)),
                  pl.BlockSpec((n, n), lambda i: (0, 0))],
        out_specs=[pl.BlockSpec((bm, n), lambda i: (i, 0)),
                   pl.BlockSpec((bm, n), lambda i: (i, 0))],
        out_shape=[jax.ShapeDtypeStruct((n, n), jnp.float32),
                   jax.ShapeDtypeStruct((n, n), jnp.float32)],
    )(jnp.full((1,), kv, jnp.int32), bg, ct)


def _gcn_body(kv_ref, g_ref, x_ref, sc_ref, w_ref, b_ref, *rest, bm, with_res):
    if with_res:
        res_ref, o_ref = rest
    else:
        (o_ref,) = rest
    kv = kv_ref[0]
    i = pl.program_id(0)
    x = x_ref[...] * sc_ref[...]
    p = jnp.dot(g_ref[...], x, preferred_element_type=jnp.float32)
    y = jax.nn.relu(jnp.dot(p, w_ref[...], preferred_element_type=jnp.float32)
                    + b_ref[...])
    if with_res:
        y = y + res_ref[...]
    rows = jax.lax.broadcasted_iota(jnp.int32, y.shape, 0) + i * bm
    o_ref[...] = jnp.where(rows < kv, y, 0.0)


def _gcn(g, x, scale, w, b, kv, res=None, bm=256):
    n = g.shape[0]
    body = functools.partial(_gcn_body, bm=bm, with_res=res is not None)
    in_specs = [pl.BlockSpec(memory_space=pltpu.SMEM),
                pl.BlockSpec((bm, n), lambda i: (i, 0)),
                pl.BlockSpec((n, D), lambda i: (0, 0)),
                pl.BlockSpec((n, 1), lambda i: (0, 0)),
                pl.BlockSpec((D, D), lambda i: (0, 0)),
                pl.BlockSpec((1, D), lambda i: (0, 0))]
    args = [jnp.full((1,), kv, jnp.int32), g, x, scale, w, b.reshape(1, D)]
    if res is not None:
        in_specs.append(pl.BlockSpec((bm, D), lambda i: (i, 0)))
        args.append(res)
    return pl.pallas_call(
        body,
        grid=(n // bm,),
        in_specs=in_specs,
        out_specs=pl.BlockSpec((bm, D), lambda i: (i, 0)),
        out_shape=jax.ShapeDtypeStruct((n, D), jnp.float32),
    )(*args)


def _gcn_final_body(g_ref, x_ref, w_ref, b_ref, res_ref, h0_ref, o1_ref, o2_ref):
    p = jnp.dot(g_ref[...], x_ref[...], preferred_element_type=jnp.float32)
    y = jax.nn.relu(jnp.dot(p, w_ref[...], preferred_element_type=jnp.float32)
                    + b_ref[...]) + res_ref[...]
    o1_ref[...] = y
    o2_ref[...] = y + h0_ref[...]


def _gcn_final(g, x, w, b, res, h0, bm=256):
    n = g.shape[0]
    return pl.pallas_call(
        _gcn_final_body,
        grid=(n // bm,),
        in_specs=[pl.BlockSpec((bm, n), lambda i: (i, 0)),
                  pl.BlockSpec((n, D), lambda i: (0, 0)),
                  pl.BlockSpec((D, D), lambda i: (0, 0)),
                  pl.BlockSpec((1, D), lambda i: (0, 0)),
                  pl.BlockSpec((bm, D), lambda i: (i, 0)),
                  pl.BlockSpec((bm, D), lambda i: (i, 0))],
        out_specs=[pl.BlockSpec((bm, D), lambda i: (i, 0)),
                   pl.BlockSpec((bm, D), lambda i: (i, 0))],
        out_shape=[jax.ShapeDtypeStruct((n, D), jnp.float32),
                   jax.ShapeDtypeStruct((n, D), jnp.float32)],
    )(g, x, w, b.reshape(1, D), res, h0)


def _pad_idx(idx):
    return jnp.concatenate([idx.astype(jnp.int32),
                            jnp.zeros((N0 - idx.shape[0],), jnp.int32)])


def kernel(g, h, Wd0, bd0, Wd1, bd1, Wb, bb, Wu0, bu0, Wu1, bu1,
           Wp0, bp0, Wp1, bp1):
    h1 = jax.nn.relu(jnp.matmul(g, h) @ Wd0 + bd0)
    s0 = jax.nn.sigmoid((h1 @ Wp0 + bp0)[:, 0])
    v0, idx0 = lax.top_k(s0, K0)
    idx0p = _pad_idx(idx0)

    a0, a0t = _maskify(g)
    b0g, c0t, newh_raw = _sc_gather3(a0, a0t, h1, idx0p)
    a1, g1 = _mask_level(b0g, c0t, K0)

    newh = newh_raw[:K0] * v0[:, None]
    g1c = g1[:K0, :K0]
    h2 = jax.nn.relu(jnp.matmul(g1c, newh) @ Wd1 + bd1)
    s1 = jax.nn.sigmoid((h2 @ Wp1 + bp1)[:, 0])
    v1k, idx1 = lax.top_k(s1, K1)
    idx1p = _pad_idx(idx1)

    a1t = _transpose(a1)
    h2p = jnp.zeros((N0, D), jnp.float32).at[:K0].set(h2)
    b1g, c1t, newh2_raw = _sc_gather3(a1, a1t, h2p, idx1p)
    _, g2 = _mask_level(b1g, c1t, K1)

    v1p = jnp.zeros((N0, 1), jnp.float32).at[:K1, 0].set(v1k)
    hb = _gcn(g2, newh2_raw, v1p, Wb, bb, K1)

    zr1 = K1
    u1idx = jnp.full((N0,), zr1, jnp.int32).at[idx1].set(
        jnp.arange(K1, dtype=jnp.int32))
    u1 = _sc_gather(hb, u1idx)

    ones = jnp.ones((N0, 1), jnp.float32)
    x1 = _gcn(g1, u1, ones, Wu0, bu0, K0, res=h2p)

    zr0 = K0
    u0idx = jnp.full((N0,), zr0, jnp.int32).at[idx0].set(
        jnp.arange(K0, dtype=jnp.int32))
    u0 = _sc_gather(x1, u0idx)

    y1, y2 = _gcn_final(g, u0, Wu1, bu1, h1, h)

    return (x1[:K0], y1, y2)


# speedup vs baseline: 2.1517x; 1.0219x over previous
import functools

import jax
import jax.numpy as jnp
from jax import lax
from jax.experimental import pallas as pl
from jax.experimental.pallas import tpu as pltpu
from jax.experimental.pallas import tpu_sc as plsc

N0 = 2048
D = 256
K0 = 1843
K1 = 1290


def _maskify_body(g_ref, a_ref, at_ref):
    a = (g_ref[...] != 0).astype(jnp.float32)
    a_ref[...] = a
    at_ref[...] = a.T


def _maskify(g, bm=256):
    n = g.shape[0]
    return pl.pallas_call(
        _maskify_body,
        grid=(n // bm, n // bm),
        in_specs=[pl.BlockSpec((bm, bm), lambda i, j: (i, j))],
        out_specs=[pl.BlockSpec((bm, bm), lambda i, j: (i, j)),
                   pl.BlockSpec((bm, bm), lambda i, j: (j, i))],
        out_shape=[jax.ShapeDtypeStruct((n, n), jnp.float32),
                   jax.ShapeDtypeStruct((n, n), jnp.float32)],
    )(g)


def _transpose_body(x_ref, o_ref):
    o_ref[...] = x_ref[...].T


def _transpose(x, bm=256):
    n = x.shape[0]
    return pl.pallas_call(
        _transpose_body,
        grid=(n // bm, n // bm),
        in_specs=[pl.BlockSpec((bm, bm), lambda i, j: (i, j))],
        out_specs=pl.BlockSpec((bm, bm), lambda i, j: (j, i)),
        out_shape=jax.ShapeDtypeStruct((n, n), x.dtype),
    )(x)


def _make_sc_gather(V, W, B):
    info = plsc.get_sparse_core_info()
    nw = info.num_cores * info.num_subcores
    num_cores = info.num_cores
    b_per_w = B // nw
    rows_per_chunk = b_per_w
    while rows_per_chunk * W > 120_000:
        rows_per_chunk //= 2
    nchunks = b_per_w // rows_per_chunk
    mesh = plsc.VectorSubcoreMesh(core_axis_name="c", subcore_axis_name="s")

    @functools.partial(
        pl.kernel, mesh=mesh,
        out_type=jax.ShapeDtypeStruct((B, W), jnp.float32),
        scratch_types=[
            pltpu.VMEM((rows_per_chunk,), jnp.int32),
            pltpu.VMEM((rows_per_chunk, W), jnp.float32),
            pltpu.SemaphoreType.DMA,
        ],
    )
    def gather_k(table_hbm, idx_hbm, out_hbm, idx_v, rows_v, sem):
        wid = lax.axis_index("s") * num_cores + lax.axis_index("c")
        base = wid * b_per_w
        for c in range(nchunks):
            off = base + c * rows_per_chunk
            pltpu.sync_copy(idx_hbm.at[pl.ds(off, rows_per_chunk)], idx_v)
            pltpu.async_copy(table_hbm.at[idx_v], rows_v, sem).wait()
            pltpu.sync_copy(rows_v, out_hbm.at[pl.ds(off, rows_per_chunk)])

    return gather_k


@functools.lru_cache(maxsize=None)
def _sc_gather_fn(V, W, B):
    return _make_sc_gather(V, W, B)


def _sc_gather(table, idx):
    V, W = table.shape
    B = idx.shape[0]
    return _sc_gather_fn(V, W, B)(table, idx)


def _make_sc_gather3(V, W, Wh, B, chunk):
    info = plsc.get_sparse_core_info()
    nw = info.num_cores * info.num_subcores
    num_cores = info.num_cores
    b_per_w = B // nw
    nchunks = b_per_w // chunk
    mesh = plsc.VectorSubcoreMesh(core_axis_name="c", subcore_axis_name="s")

    @functools.partial(
        pl.kernel, mesh=mesh,
        out_type=[jax.ShapeDtypeStruct((B, W), jnp.float32),
                  jax.ShapeDtypeStruct((B, W), jnp.float32),
                  jax.ShapeDtypeStruct((B, Wh), jnp.float32)],
        scratch_types=[
            pltpu.VMEM((b_per_w,), jnp.int32),
            pltpu.VMEM((2, chunk, W), jnp.float32),
            pltpu.VMEM((2, chunk, W), jnp.float32),
            pltpu.VMEM((b_per_w, Wh), jnp.float32),
            pltpu.SemaphoreType.DMA,
            pltpu.SemaphoreType.DMA,
            pltpu.SemaphoreType.DMA,
            pltpu.SemaphoreType.DMA,
            pltpu.SemaphoreType.DMA,
        ],
    )
    def gather_k(t0_hbm, t1_hbm, th_hbm, idx_hbm, o0_hbm, o1_hbm, oh_hbm,
                 idx_v, r0, r1, rh, s0a, s0b, s1a, s1b, sh):
        wid = lax.axis_index("s") * num_cores + lax.axis_index("c")
        base = wid * b_per_w
        s0 = (s0a, s0b)
        s1 = (s1a, s1b)
        pltpu.sync_copy(idx_hbm.at[pl.ds(base, b_per_w)], idx_v)
        ch = pltpu.async_copy(th_hbm.at[idx_v], rh, sh)
        cps = [None] * nchunks
        for c in range(nchunks):
            sl = pl.ds(c * chunk, chunk)
            b = c % 2
            cps[c] = (pltpu.async_copy(t0_hbm.at[idx_v.at[sl]], r0.at[b], s0[b]),
                      pltpu.async_copy(t1_hbm.at[idx_v.at[sl]], r1.at[b], s1[b]))
            if c > 0:
                off = base + (c - 1) * chunk
                pb = (c - 1) % 2
                cps[c - 1][0].wait()
                pltpu.sync_copy(r0.at[pb], o0_hbm.at[pl.ds(off, chunk)])
                cps[c - 1][1].wait()
                pltpu.sync_copy(r1.at[pb], o1_hbm.at[pl.ds(off, chunk)])
        off = base + (nchunks - 1) * chunk
        pb = (nchunks - 1) % 2
        cps[nchunks - 1][0].wait()
        pltpu.sync_copy(r0.at[pb], o0_hbm.at[pl.ds(off, chunk)])
        cps[nchunks - 1][1].wait()
        pltpu.sync_copy(r1.at[pb], o1_hbm.at[pl.ds(off, chunk)])
        ch.wait()
        pltpu.sync_copy(rh, oh_hbm.at[pl.ds(base, b_per_w)])

    return gather_k


@functools.lru_cache(maxsize=None)
def _sc_gather3_fn(V, W, Wh, B, chunk):
    return _make_sc_gather3(V, W, Wh, B, chunk)


def _sc_gather3(t0, t1, th, idx, chunk=8):
    V, W = t0.shape
    Wh = th.shape[1]
    B = idx.shape[0]
    return _sc_gather3_fn(V, W, Wh, B, chunk)(t0, t1, th, idx)


def _mask_body(kv_ref, b_ref, c_ref, m_ref, gn_ref, *, bm):
    kv = kv_ref[0]
    i = pl.program_id(0)
    s = lax.dot_general(b_ref[...], c_ref[...], (((1,), (1,)), ((), ())),
                        preferred_element_type=jnp.float32)
    rows = jax.lax.broadcasted_iota(jnp.int32, s.shape, 0) + i * bm
    cols = jax.lax.broadcasted_iota(jnp.int32, s.shape, 1)
    valid = (rows < kv) & (cols < kv)
    m = jnp.where((s > 0) & valid, 1.0, 0.0)
    m_ref[...] = m
    deg = jnp.sum(m, axis=1, keepdims=True)
    gn_ref[...] = m / jnp.maximum(deg, 1.0)


def _mask_level(bg, ct, kv, bm=256):
    n = bg.shape[0]
    body = functools.partial(_mask_body, bm=bm)
    return pl.pallas_call(
        body,
        grid=(n // bm,),
        in_specs=[pl.BlockSpec(memory_space=pltpu.SMEM),
                  pl.BlockSpec((bm, n), lambda i: (i, 0)),
                  pl.BlockSpec((n, n), lambda i: (0, 0))],
        out_specs=[pl.BlockSpec((bm, n), lambda i: (i, 0)),
                   pl.BlockSpec((bm, n), lambda i: (i, 0))],
        out_shape=[jax.ShapeDtypeStruct((n, n), jnp.float32),
                   jax.ShapeDtypeStruct((n, n), jnp.float32)],
    )(jnp.full((1,), kv, jnp.int32), bg, ct)


def _gcn_body(kv_ref, g_ref, x_ref, sc_ref, w_ref, b_ref, *rest, bm, with_res):
    if with_res:
        res_ref, o_ref = rest
    else:
        (o_ref,) = rest
    kv = kv_ref[0]
    i = pl.program_id(0)
    x = x_ref[...] * sc_ref[...]
    p = jnp.dot(g_ref[...], x, preferred_element_type=jnp.float32)
    y = jax.nn.relu(jnp.dot(p, w_ref[...], preferred_element_type=jnp.float32)
                    + b_ref[...])
    if with_res:
        y = y + res_ref[...]
    rows = jax.lax.broadcasted_iota(jnp.int32, y.shape, 0) + i * bm
    o_ref[...] = jnp.where(rows < kv, y, 0.0)


def _gcn(g, x, scale, w, b, kv, res=None, bm=256):
    n = g.shape[0]
    body = functools.partial(_gcn_body, bm=bm, with_res=res is not None)
    in_specs = [pl.BlockSpec(memory_space=pltpu.SMEM),
                pl.BlockSpec((bm, n), lambda i: (i, 0)),
                pl.BlockSpec((n, D), lambda i: (0, 0)),
                pl.BlockSpec((n, 1), lambda i: (0, 0)),
                pl.BlockSpec((D, D), lambda i: (0, 0)),
                pl.BlockSpec((1, D), lambda i: (0, 0))]
    args = [jnp.full((1,), kv, jnp.int32), g, x, scale, w, b.reshape(1, D)]
    if res is not None:
        in_specs.append(pl.BlockSpec((bm, D), lambda i: (i, 0)))
        args.append(res)
    return pl.pallas_call(
        body,
        grid=(n // bm,),
        in_specs=in_specs,
        out_specs=pl.BlockSpec((bm, D), lambda i: (i, 0)),
        out_shape=jax.ShapeDtypeStruct((n, D), jnp.float32),
    )(*args)


def _gcn_final_body(g_ref, x_ref, w_ref, b_ref, res_ref, h0_ref, o1_ref, o2_ref):
    p = jnp.dot(g_ref[...], x_ref[...], preferred_element_type=jnp.float32)
    y = jax.nn.relu(jnp.dot(p, w_ref[...], preferred_element_type=jnp.float32)
                    + b_ref[...]) + res_ref[...]
    o1_ref[...] = y
    o2_ref[...] = y + h0_ref[...]


def _gcn_final(g, x, w, b, res, h0, bm=256):
    n = g.shape[0]
    return pl.pallas_call(
        _gcn_final_body,
        grid=(n // bm,),
        in_specs=[pl.BlockSpec((bm, n), lambda i: (i, 0)),
                  pl.BlockSpec((n, D), lambda i: (0, 0)),
                  pl.BlockSpec((D, D), lambda i: (0, 0)),
                  pl.BlockSpec((1, D), lambda i: (0, 0)),
                  pl.BlockSpec((bm, D), lambda i: (i, 0)),
                  pl.BlockSpec((bm, D), lambda i: (i, 0))],
        out_specs=[pl.BlockSpec((bm, D), lambda i: (i, 0)),
                   pl.BlockSpec((bm, D), lambda i: (i, 0))],
        out_shape=[jax.ShapeDtypeStruct((n, D), jnp.float32),
                   jax.ShapeDtypeStruct((n, D), jnp.float32)],
    )(g, x, w, b.reshape(1, D), res, h0)


def _pad_idx(idx):
    return jnp.concatenate([idx.astype(jnp.int32),
                            jnp.zeros((N0 - idx.shape[0],), jnp.int32)])


def kernel(g, h, Wd0, bd0, Wd1, bd1, Wb, bb, Wu0, bu0, Wu1, bu1,
           Wp0, bp0, Wp1, bp1):
    h1 = jax.nn.relu(jnp.matmul(g, h) @ Wd0 + bd0)
    s0 = jax.nn.sigmoid((h1 @ Wp0 + bp0)[:, 0])
    v0, idx0 = lax.top_k(s0, K0)
    idx0p = _pad_idx(idx0)

    a0, a0t = _maskify(g)
    b0g, c0t, newh_raw = _sc_gather3(a0, a0t, h1, idx0p)
    a1, g1 = _mask_level(b0g, c0t, K0)

    newh = newh_raw[:K0] * v0[:, None]
    g1c = g1[:K0, :K0]
    h2 = jax.nn.relu(jnp.matmul(g1c, newh) @ Wd1 + bd1)
    s1 = jax.nn.sigmoid((h2 @ Wp1 + bp1)[:, 0])
    v1k, idx1 = lax.top_k(s1, K1)
    idx1p = _pad_idx(idx1)

    a1t = _transpose(a1)
    h2p = jnp.zeros((N0, D), jnp.float32).at[:K0].set(h2)
    b1g, c1t, newh2_raw = _sc_gather3(a1, a1t, h2p, idx1p)
    _, g2 = _mask_level(b1g, c1t, K1)

    v1p = jnp.zeros((N0, 1), jnp.float32).at[:K1, 0].set(v1k)
    hb = _gcn(g2, newh2_raw, v1p, Wb, bb, K1)

    zr1 = K1
    u1idx = jnp.full((N0,), zr1, jnp.int32).at[idx1].set(
        jnp.arange(K1, dtype=jnp.int32))
    u1 = _sc_gather(hb, u1idx)

    ones = jnp.ones((N0, 1), jnp.float32)
    x1 = _gcn(g1, u1, ones, Wu0, bu0, K0, res=h2p)

    zr0 = K0
    u0idx = jnp.full((N0,), zr0, jnp.int32).at[idx0].set(
        jnp.arange(K0, dtype=jnp.int32))
    u0 = _sc_gather(x1, u0idx)

    y1, y2 = _gcn_final(g, u0, Wu1, bu1, h1, h)

    return (x1[:K0], y1, y2)
